# baseline (speedup 1.0000x reference)
import functools

import jax
import jax.numpy as jnp
from jax import lax
from jax.experimental import pallas as pl
from jax.experimental.pallas import tpu as pltpu
from jax.experimental.pallas import tpu_sc as plsc

BLK = 8192
CH = 256
NW = 32
INT_MIN = -2147483648


def _gate_body(x_ref, wp_ref, bp_ref, wg_ref, bg_ref, y_ref):
    x = x_ref[...]
    p = jnp.dot(x, wp_ref[...], preferred_element_type=jnp.float32) + bp_ref[...]
    g = jnp.dot(x, wg_ref[...], preferred_element_type=jnp.float32) + bg_ref[...]
    y_ref[...] = jax.nn.sigmoid(g) * p


def _finish_body(part_ref, wf_ref, bf_ref, o_ref):
    s = jnp.sum(part_ref[...], axis=0)
    o_ref[...] = jnp.dot(s, wf_ref[...], preferred_element_type=jnp.float32) + bf_ref[...]


def _sc_pool(num_groups, h, v, bnd_pad, y_hbm, bnd_hbm, out_hbm, bnd_v, buf_v, acc_v):
    nlanes = 16
    rows_per_tile = v // NW
    wid = lax.axis_index("s") * 2 + lax.axis_index("c")
    base = wid * rows_per_tile

    zeros = jnp.zeros((nlanes,), jnp.float32)
    for g in range(num_groups):
        for j in range(h // nlanes):
            acc_v[g, pl.ds(j * nlanes, nlanes)] = zeros

    pltpu.sync_copy(bnd_hbm, bnd_v)
    lov = bnd_v[pl.ds(0, nlanes)]
    hiv = bnd_v[pl.ds(8, nlanes)]
    sb = [lov[g] for g in range(nlanes)] + [hiv[g - 8] for g in range(nlanes, num_groups + 1)]

    for c in range(rows_per_tile // CH):
        cb = base + c * CH
        pltpu.sync_copy(y_hbm.at[pl.ds(cb, CH), :], buf_v)
        for g in range(num_groups):
            l0 = jnp.maximum(sb[g] - cb, 0)
            h0 = jnp.minimum(sb[g + 1] - 1 - cb, CH)
            h0 = jnp.maximum(h0, l0)

            def row_add(r, carry):
                return tuple(
                    carry[j] + buf_v[r, pl.ds(j * nlanes, nlanes)]
                    for j in range(h // nlanes)
                )

            init = tuple(jnp.zeros((nlanes,), jnp.float32)
                         for _ in range(h // nlanes))
            summed = lax.fori_loop(l0, h0, row_add, init)
            for j in range(h // nlanes):
                plsc.addupdate(acc_v.at[g, pl.ds(j * nlanes, nlanes)], summed[j])

    pltpu.sync_copy(acc_v, out_hbm.at[wid])


def kernel(node_features, node_grp_start_with_end, max_size, Wp, bp, Wg, bg, Wf, bf):
    v, h = node_features.shape
    g = node_grp_start_with_end.shape[0] - 1
    hp = Wp.shape[1]
    ho = Wf.shape[1]
    num_blocks = v // BLK

    y = pl.pallas_call(
        _gate_body,
        grid=(num_blocks,),
        in_specs=[
            pl.BlockSpec((BLK, h), lambda i: (i, 0)),
            pl.BlockSpec((h, hp), lambda i: (0, 0)),
            pl.BlockSpec((1, hp), lambda i: (0, 0)),
            pl.BlockSpec((h, hp), lambda i: (0, 0)),
            pl.BlockSpec((1, hp), lambda i: (0, 0)),
        ],
        out_specs=pl.BlockSpec((BLK, hp), lambda i: (i, 0)),
        out_shape=jax.ShapeDtypeStruct((v, hp), jnp.float32),
    )(node_features, Wp, bp.reshape(1, hp), Wg, bg.reshape(1, hp))

    bnd_pad = jnp.pad(node_grp_start_with_end.astype(jnp.int32), (0, 15),
                      constant_values=v)

    mesh = plsc.VectorSubcoreMesh(core_axis_name="c", subcore_axis_name="s")
    sc_pool = functools.partial(
        pl.kernel,
        mesh=mesh,
        out_type=jax.ShapeDtypeStruct((NW, g, hp), jnp.float32),
        scratch_types=[
            pltpu.VMEM((32,), jnp.int32),
            pltpu.VMEM((CH, hp), jnp.float32),
            pltpu.VMEM((g, hp), jnp.float32),
        ],
    )(functools.partial(_sc_pool, g, hp, v, None))
    partials = sc_pool(y, bnd_pad)

    out = pl.pallas_call(
        _finish_body,
        in_specs=[
            pl.BlockSpec((NW, g, hp), lambda: (0, 0, 0)),
            pl.BlockSpec((hp, ho), lambda: (0, 0)),
            pl.BlockSpec((1, ho), lambda: (0, 0)),
        ],
        out_specs=pl.BlockSpec((g, ho), lambda: (0, 0)),
        out_shape=jax.ShapeDtypeStruct((g, ho), jnp.float32),
    )(partials, Wf, bf.reshape(1, ho))
    return out


# speedup vs baseline: 4.5847x; 4.5847x over previous
import functools

import jax
import jax.numpy as jnp
from jax.experimental import pallas as pl
from jax.experimental.pallas import tpu as pltpu

BLK = 8192


def _body(starts_ref, x_ref, wp_ref, bp_ref, wg_ref, bg_ref, wf_ref, bf_ref,
          o_ref, acc_ref, *, num_blocks, num_groups):
    i = pl.program_id(0)
    x = x_ref[...]
    p = jnp.dot(x, wp_ref[...], preferred_element_type=jnp.float32) + bp_ref[...]
    g = jnp.dot(x, wg_ref[...], preferred_element_type=jnp.float32) + bg_ref[...]
    y = jax.nn.sigmoid(g) * p

    cols = i * BLK + jax.lax.broadcasted_iota(jnp.int32, (num_groups, BLK), 1)
    gidx = jax.lax.broadcasted_iota(jnp.int32, (num_groups, 1), 0)
    lo = jnp.zeros((num_groups, 1), jnp.int32)
    hi = jnp.zeros((num_groups, 1), jnp.int32)
    for s in range(num_groups):
        lo = jnp.where(gidx == s, starts_ref[s], lo)
        hi = jnp.where(gidx == s, starts_ref[s + 1], hi)
    a = jnp.where(jnp.logical_and(cols >= lo, cols <= hi - 2), 1.0, 0.0)
    partial = jnp.dot(a, y, preferred_element_type=jnp.float32)

    @pl.when(i == 0)
    def _init():
        acc_ref[...] = jnp.zeros_like(acc_ref)

    acc_ref[...] += partial

    @pl.when(i == num_blocks - 1)
    def _finish():
        o_ref[...] = (
            jnp.dot(acc_ref[...], wf_ref[...], preferred_element_type=jnp.float32)
            + bf_ref[...]
        )


def kernel(node_features, node_grp_start_with_end, max_size, Wp, bp, Wg, bg, Wf, bf):
    v, h = node_features.shape
    g = node_grp_start_with_end.shape[0] - 1
    hp = Wp.shape[1]
    ho = Wf.shape[1]
    num_blocks = v // BLK

    grid_spec = pltpu.PrefetchScalarGridSpec(
        num_scalar_prefetch=1,
        grid=(num_blocks,),
        in_specs=[
            pl.BlockSpec((BLK, h), lambda i, s: (i, 0)),
            pl.BlockSpec((h, hp), lambda i, s: (0, 0)),
            pl.BlockSpec((1, hp), lambda i, s: (0, 0)),
            pl.BlockSpec((h, hp), lambda i, s: (0, 0)),
            pl.BlockSpec((1, hp), lambda i, s: (0, 0)),
            pl.BlockSpec((hp, ho), lambda i, s: (0, 0)),
            pl.BlockSpec((1, ho), lambda i, s: (0, 0)),
        ],
        out_specs=pl.BlockSpec((g, ho), lambda i, s: (0, 0)),
        scratch_shapes=[pltpu.VMEM((g, hp), jnp.float32)],
    )

    out = pl.pallas_call(
        functools.partial(_body, num_blocks=num_blocks, num_groups=g),
        grid_spec=grid_spec,
        out_shape=jax.ShapeDtypeStruct((g, ho), jnp.float32),
    )(
        node_grp_start_with_end,
        node_features,
        Wp, bp.reshape(1, hp),
        Wg, bg.reshape(1, hp),
        Wf, bf.reshape(1, ho),
    )
    return out
